# baseline (speedup 1.0000x reference)
import functools

import jax
import jax.numpy as jnp
from jax import lax
from jax.experimental import pallas as pl
from jax.experimental.pallas import tpu as pltpu
from jax.experimental.pallas import tpu_sc as plsc

_N = 1 << 24
_OUT = _N >> 1
_B = 20
_NW = 32
_PER_W = _OUT // _NW
_W_PER_CHUNK = (1 << _B) // _PER_W

_mesh = plsc.VectorSubcoreMesh(core_axis_name="c", subcore_axis_name="s")


_BUF = 16384
_NBUF = 4
_STEPS = _PER_W // _BUF


@functools.partial(
    pl.kernel,
    mesh=_mesh,
    out_type=jax.ShapeDtypeStruct((_OUT,), jnp.float32),
    scratch_types=(
        [pltpu.VMEM((_BUF,), jnp.float32)] * _NBUF
        + [pltpu.SemaphoreType.DMA] * (2 * _NBUF)
    ),
)
def _restrict(psi_hbm, out_hbm, *scratch):
    bufs = scratch[:_NBUF]
    isems = scratch[_NBUF:2 * _NBUF]
    osems = scratch[2 * _NBUF:]
    wid = lax.axis_index("s") * 2 + lax.axis_index("c")
    out_base = wid * _PER_W
    in_base = (wid // _W_PER_CHUNK) * (1 << (_B + 1)) + (wid % _W_PER_CHUNK) * _PER_W

    def start_in(step, b):
        return pltpu.async_copy(
            psi_hbm.at[pl.ds(in_base + step * _BUF, _BUF)], bufs[b], isems[b])

    def start_out(step, b):
        return pltpu.async_copy(
            bufs[b], out_hbm.at[pl.ds(out_base + step * _BUF, _BUF)], osems[b])

    in_cp = [None] * _NBUF
    out_cp = [None] * _NBUF
    for j in range(min(_NBUF - 1, _STEPS)):
        in_cp[j] = start_in(j, j)
    for i in range(_STEPS):
        b = i % _NBUF
        nxt = i + _NBUF - 1
        if nxt < _STEPS:
            nb = nxt % _NBUF
            if out_cp[nb] is not None:
                out_cp[nb].wait()
            in_cp[nb] = start_in(nxt, nb)
        in_cp[b].wait()
        out_cp[b] = start_out(i, b)
    for b in range(_NBUF):
        if out_cp[b] is not None:
            out_cp[b].wait()


def kernel(psi):
    return _restrict(psi)


# speedup vs baseline: 1.0105x; 1.0105x over previous
import functools

import jax
import jax.numpy as jnp
from jax import lax
from jax.experimental import pallas as pl
from jax.experimental.pallas import tpu as pltpu
from jax.experimental.pallas import tpu_sc as plsc

_N = 1 << 24
_OUT = _N >> 1
_B = 20
_NW = 32
_PER_W = _OUT // _NW
_W_PER_CHUNK = (1 << _B) // _PER_W

_BUF = 49152
_NBUF = 2
_SIZES = [49152] * 5 + [16384]
_OFFS = [sum(_SIZES[:i]) for i in range(len(_SIZES))]
assert sum(_SIZES) == _PER_W and all(s <= _BUF for s in _SIZES)
_STEPS = len(_SIZES)

_mesh = plsc.VectorSubcoreMesh(core_axis_name="c", subcore_axis_name="s")


@functools.partial(
    pl.kernel,
    mesh=_mesh,
    out_type=jax.ShapeDtypeStruct((_OUT,), jnp.float32),
    scratch_types=(
        [pltpu.VMEM((_BUF,), jnp.float32)] * _NBUF
        + [pltpu.SemaphoreType.DMA] * (2 * _NBUF)
    ),
)
def _restrict(psi_hbm, out_hbm, *scratch):
    bufs = scratch[:_NBUF]
    isems = scratch[_NBUF:2 * _NBUF]
    osems = scratch[2 * _NBUF:]
    wid = lax.axis_index("s") * 2 + lax.axis_index("c")
    out_base = wid * _PER_W
    in_base = (wid // _W_PER_CHUNK) * (1 << (_B + 1)) + (wid % _W_PER_CHUNK) * _PER_W

    def start_in(step, b):
        sz = _SIZES[step]
        return pltpu.async_copy(
            psi_hbm.at[pl.ds(in_base + _OFFS[step], sz)],
            bufs[b].at[pl.ds(0, sz)], isems[b])

    def start_out(step, b):
        sz = _SIZES[step]
        return pltpu.async_copy(
            bufs[b].at[pl.ds(0, sz)],
            out_hbm.at[pl.ds(out_base + _OFFS[step], sz)], osems[b])

    in_cp = [None] * _NBUF
    out_cp = [None] * _NBUF
    for j in range(min(_NBUF - 1, _STEPS)):
        in_cp[j] = start_in(j, j)
    for i in range(_STEPS):
        b = i % _NBUF
        nxt = i + _NBUF - 1
        if nxt < _STEPS:
            nb = nxt % _NBUF
            if out_cp[nb] is not None:
                out_cp[nb].wait()
            in_cp[nb] = start_in(nxt, nb)
        in_cp[b].wait()
        out_cp[b] = start_out(i, b)
    for b in range(_NBUF):
        if out_cp[b] is not None:
            out_cp[b].wait()


def kernel(psi):
    return _restrict(psi)


# speedup vs baseline: 1.0242x; 1.0136x over previous
import functools

import jax
import jax.numpy as jnp
from jax import lax
from jax.experimental import pallas as pl
from jax.experimental.pallas import tpu as pltpu
from jax.experimental.pallas import tpu_sc as plsc

_N = 1 << 24
_OUT = _N >> 1
_B = 20
_NW = 32
_PER_W = _OUT // _NW
_W_PER_CHUNK = (1 << _B) // _PER_W

_mesh = plsc.VectorSubcoreMesh(core_axis_name="c", subcore_axis_name="s")


_BUF = 32768
_NBUF = 3
_STEPS = _PER_W // _BUF


@functools.partial(
    pl.kernel,
    mesh=_mesh,
    out_type=jax.ShapeDtypeStruct((_OUT,), jnp.float32),
    scratch_types=(
        [pltpu.VMEM((_BUF,), jnp.float32)] * _NBUF
        + [pltpu.SemaphoreType.DMA] * (2 * _NBUF)
    ),
)
def _restrict(psi_hbm, out_hbm, *scratch):
    bufs = scratch[:_NBUF]
    isems = scratch[_NBUF:2 * _NBUF]
    osems = scratch[2 * _NBUF:]
    wid = lax.axis_index("s") * 2 + lax.axis_index("c")
    out_base = wid * _PER_W
    in_base = (wid // _W_PER_CHUNK) * (1 << (_B + 1)) + (wid % _W_PER_CHUNK) * _PER_W

    def start_in(step, b):
        return pltpu.async_copy(
            psi_hbm.at[pl.ds(in_base + step * _BUF, _BUF)], bufs[b], isems[b])

    def start_out(step, b):
        return pltpu.async_copy(
            bufs[b], out_hbm.at[pl.ds(out_base + step * _BUF, _BUF)], osems[b])

    in_cp = [None] * _NBUF
    out_cp = [None] * _NBUF
    for j in range(min(_NBUF - 1, _STEPS)):
        in_cp[j] = start_in(j, j)
    for i in range(_STEPS):
        b = i % _NBUF
        nxt = i + _NBUF - 1
        if nxt < _STEPS:
            nb = nxt % _NBUF
            if out_cp[nb] is not None:
                out_cp[nb].wait()
            in_cp[nb] = start_in(nxt, nb)
        in_cp[b].wait()
        out_cp[b] = start_out(i, b)
    for b in range(_NBUF):
        if out_cp[b] is not None:
            out_cp[b].wait()


def kernel(psi):
    return _restrict(psi)
